# Initial kernel scaffold; baseline (speedup 1.0000x reference)
import jax
import jax.numpy as jnp
from jax.experimental import pallas as pl


def kernel(adj_index, adj_values, image_adj_index, image_adj_values, text_adj_index, text_adj_values, image_embedding, text_embedding, u_embs, i_embs, W_img, b_img, W_txt, b_txt, modal_weight):
    raise NotImplementedError("write your pallas kernel here")


# speedup vs baseline: 3.6174x; 3.6174x over previous
import functools

import jax
import jax.numpy as jnp
from jax import lax
from jax.experimental import pallas as pl
from jax.experimental.pallas import tpu as pltpu
from jax.experimental.pallas import tpu_sc as plsc

USER = 6000
ITEM = 4000
NN = USER + ITEM
EDGES = 320000
D = 128
MODAL_ADJ_WEIGHT = 0.4
RESIDUAL_WEIGHT = 0.5

CH = 128
ZR = 208
NC = 2
NS = 16
NW = NC * NS
ROWS_PER_TILE = 624


def _l2n(x):
    n = jnp.sqrt(jnp.sum(x * x, axis=-1, keepdims=True))
    return x / jnp.maximum(n, 1e-12)


def _feats_body(img_ref, wimg_ref, bimg_ref, txt_ref, wtxt_ref, btxt_ref,
                u_ref, i_ref, mw_ref, vadj_ref, vimg_ref, vtxt_ref,
                t_ref, vals_ref):
    img_f = jnp.dot(img_ref[...], wimg_ref[...],
                    preferred_element_type=jnp.float32) + bimg_ref[...]
    txt_f = jnp.dot(txt_ref[...], wtxt_ref[...],
                    preferred_element_type=jnp.float32) + btxt_ref[...]
    img_n = _l2n(img_f)
    txt_n = _l2n(txt_f)
    u = u_ref[...]
    mw = mw_ref[...]
    m = jnp.exp(mw - jnp.max(mw))
    w = m / jnp.sum(m)
    w0 = w[0, 0]
    w1 = w[0, 1]
    t_ref[0:USER] = u
    t_ref[USER:NN] = i_ref[...]
    t_ref[NN:NN + USER] = u
    t_ref[NN + USER:2 * NN] = img_n
    t_ref[2 * NN:2 * NN + USER] = u
    t_ref[2 * NN + USER:3 * NN] = txt_n
    nb = EDGES // D
    vals_ref[0:nb] = vadj_ref[...]
    vals_ref[nb:2 * nb] = vimg_ref[...] * (MODAL_ADJ_WEIGHT * w0)
    vals_ref[2 * nb:3 * nb] = vtxt_ref[...] * (MODAL_ADJ_WEIGHT * w1)


_feats = pl.pallas_call(
    _feats_body,
    out_shape=(
        jax.ShapeDtypeStruct((3 * NN, D), jnp.float32),
        jax.ShapeDtypeStruct((3 * EDGES // D, D), jnp.float32),
    ),
)


def _sum2_body(p_ref, o_ref):
    o_ref[...] = p_ref[0] + p_ref[1]


_sum2 = pl.pallas_call(
    _sum2_body,
    out_shape=jax.ShapeDtypeStruct((NN, D), jnp.float32),
)


def _final_body(modal_ref, q_ref, o_ref):
    o_ref[...] = (1.0 + RESIDUAL_WEIGHT) * modal_ref[...] + q_ref[0] + q_ref[1]


_final = pl.pallas_call(
    _final_body,
    out_shape=jax.ShapeDtypeStruct((NN, D), jnp.float32),
)


@functools.cache
def _make_spmm(n_edges):
    n_chunks = n_edges // CH
    base_chunks = n_chunks // NW
    extra = n_chunks % NW
    mesh = plsc.VectorSubcoreMesh(core_axis_name="c", subcore_axis_name="s")

    @functools.partial(
        pl.kernel,
        out_type=jax.ShapeDtypeStruct((NC, NN, D), jnp.float32),
        mesh=mesh,
        compiler_params=pltpu.CompilerParams(needs_layout_passes=False),
        scratch_types=[
            pltpu.VMEM((CH,), jnp.int32),
            pltpu.VMEM((CH,), jnp.int32),
            pltpu.VMEM((CH,), jnp.float32),
            pltpu.VMEM((CH, D), jnp.float32),
            pltpu.VMEM((ZR, D), jnp.float32),
            pltpu.VMEM_SHARED((NN, D), jnp.float32),
            pltpu.SemaphoreType.DMA,
        ],
    )
    def spmm(row_hbm, col_hbm, val_hbm, x_hbm, out_hbm,
             colv, rowv, valv, rowsbuf, zbuf, acc, sem):
        cid = lax.axis_index("c")
        sid = lax.axis_index("s")
        wid = sid * NC + cid

        def zero_zbuf(i, carry):
            for g in range(D // 16):
                zbuf[i, pl.ds(g * 16, 16)] = jnp.zeros((16,), jnp.float32)
            return carry

        lax.fori_loop(0, ZR, zero_zbuf, 0)
        r0 = sid * ROWS_PER_TILE
        for k in range(ROWS_PER_TILE // ZR):
            pltpu.sync_copy(zbuf, acc.at[pl.ds(r0 + k * ZR, ZR)])

        @pl.when(sid == NS - 1)
        def _zero_tail():
            pltpu.sync_copy(zbuf.at[pl.ds(0, NN - NS * ROWS_PER_TILE)],
                            acc.at[pl.ds(NS * ROWS_PER_TILE,
                                         NN - NS * ROWS_PER_TILE)])

        plsc.subcore_barrier()

        nch = base_chunks + jnp.where(wid < extra, 1, 0)

        def chunk_body(k, carry):
            base = (wid + k * NW) * CH
            pltpu.sync_copy(col_hbm.at[pl.ds(base, CH)], colv)
            pltpu.sync_copy(row_hbm.at[pl.ds(base, CH)], rowv)
            pltpu.sync_copy(val_hbm.at[pl.ds(base, CH)], valv)
            pltpu.async_copy(x_hbm.at[colv], rowsbuf, sem).wait()

            def scale_edge(e, c2):
                v = plsc.load_gather(valv, [jnp.full((16,), e, jnp.int32)])
                for g in range(D // 16):
                    sl = pl.ds(g * 16, 16)
                    rowsbuf[e, sl] = rowsbuf[e, sl] * v
                return c2

            lax.fori_loop(0, CH, scale_edge, 0)
            pltpu.sync_copy(rowsbuf, acc.at[rowv], add=True)
            return carry

        lax.fori_loop(0, nch, chunk_body, 0)
        plsc.subcore_barrier()
        for k in range(ROWS_PER_TILE // ZR):
            rr = r0 + k * ZR
            pltpu.sync_copy(acc.at[pl.ds(rr, ZR)], out_hbm.at[cid, pl.ds(rr, ZR)])

        @pl.when(sid == NS - 1)
        def _write_tail():
            t0 = NS * ROWS_PER_TILE
            tn = NN - t0
            pltpu.sync_copy(acc.at[pl.ds(t0, tn)], out_hbm.at[cid, pl.ds(t0, tn)])

    return spmm


def kernel(adj_index, adj_values, image_adj_index, image_adj_values,
           text_adj_index, text_adj_values, image_embedding, text_embedding,
           u_embs, i_embs, W_img, b_img, W_txt, b_txt, modal_weight):
    ai = adj_index.astype(jnp.int32)
    ii = image_adj_index.astype(jnp.int32)
    ti = text_adj_index.astype(jnp.int32)
    rows_all = jnp.concatenate([ai[0], ii[0], ti[0]])
    cols_all = jnp.concatenate([ai[1], ii[1] + NN, ti[1] + 2 * NN])

    table, vals2d = _feats(
        image_embedding, W_img, b_img.reshape(1, D),
        text_embedding, W_txt, b_txt.reshape(1, D),
        u_embs, i_embs, modal_weight.reshape(1, 2),
        adj_values.reshape(-1, D), image_adj_values.reshape(-1, D),
        text_adj_values.reshape(-1, D),
    )
    vals_all = vals2d.reshape(-1)

    p = _make_spmm(3 * EDGES)(rows_all, cols_all, vals_all, table)
    modal = _sum2(p)
    q = _make_spmm(EDGES)(ai[0], ai[1], adj_values, modal)
    return _final(modal, q)
